# baseline (speedup 1.0000x reference)
import functools

import jax
import jax.numpy as jnp
from jax import lax
from jax.experimental import pallas as pl
from jax.experimental.pallas import tpu as pltpu
from jax.experimental.pallas import tpu_sc as plsc

VOCAB = 100000
EMBED_DIM = 16
BATCH = 1024


_NC = 2
_NS = 16
_NW = _NC * _NS
_B_PER_W = BATCH // _NW


def _sc_gather(idx_hbm, table_hbm, out_hbm, idx_v, rows_v, sem):
    wid = lax.axis_index("s") * _NC + lax.axis_index("c")
    base = wid * _B_PER_W
    pltpu.sync_copy(idx_hbm.at[pl.ds(base, _B_PER_W)], idx_v)
    pltpu.async_copy(table_hbm.at[idx_v], rows_v, sem).wait()
    pltpu.sync_copy(rows_v, out_hbm.at[pl.ds(base, _B_PER_W)])


@functools.cache
def _gather_kernel():
    return pl.kernel(
        _sc_gather,
        mesh=plsc.VectorSubcoreMesh(core_axis_name="c", subcore_axis_name="s"),
        out_type=jax.ShapeDtypeStruct((BATCH, EMBED_DIM), jnp.float32),
        scratch_types=[
            pltpu.VMEM((_B_PER_W,), jnp.int32),
            pltpu.VMEM((_B_PER_W, EMBED_DIM), jnp.float32),
            pltpu.SemaphoreType.DMA,
        ],
        compiler_params=pltpu.CompilerParams(use_tc_tiling_on_sc=False),
    )


_VBLK = 2048


def _proj_body(latent_ref, w_ref, b_ref, out_ref):
    out_ref[...] = lax.dot_general(
        latent_ref[...], w_ref[...],
        dimension_numbers=(((1,), (1,)), ((), ())),
        preferred_element_type=jnp.float32,
    ) + b_ref[...]


def _project(latent, W, b2d):
    grid = (pl.cdiv(VOCAB, _VBLK),)
    return pl.pallas_call(
        _proj_body,
        grid=grid,
        in_specs=[
            pl.BlockSpec((BATCH, EMBED_DIM), lambda j: (0, 0)),
            pl.BlockSpec((_VBLK, EMBED_DIM), lambda j: (j, 0)),
            pl.BlockSpec((1, _VBLK), lambda j: (0, j)),
        ],
        out_specs=pl.BlockSpec((BATCH, _VBLK), lambda j: (0, j)),
        out_shape=jax.ShapeDtypeStruct((BATCH, VOCAB), jnp.float32),
        compiler_params=pltpu.CompilerParams(
            dimension_semantics=("arbitrary",),
        ),
    )(latent, W, b2d)


def kernel(inputs, emb_table, W, b):
    idx = inputs.astype(jnp.int32)
    latent = _gather_kernel()(idx, emb_table)
    return _project(latent, W, b.reshape(1, VOCAB))


# speedup vs baseline: 1.0273x; 1.0273x over previous
import functools

import jax
import jax.numpy as jnp
from jax import lax
from jax.experimental import pallas as pl
from jax.experimental.pallas import tpu as pltpu
from jax.experimental.pallas import tpu_sc as plsc

VOCAB = 100000
EMBED_DIM = 16
BATCH = 1024


_NC = 2
_NS = 16
_NW = _NC * _NS
_B_PER_W = BATCH // _NW


def _sc_gather(idx_hbm, table_hbm, out_hbm, idx_v, rows_v, sem):
    wid = lax.axis_index("s") * _NC + lax.axis_index("c")
    base = wid * _B_PER_W
    pltpu.sync_copy(idx_hbm.at[pl.ds(base, _B_PER_W)], idx_v)
    pltpu.async_copy(table_hbm.at[idx_v], rows_v, sem).wait()
    pltpu.sync_copy(rows_v, out_hbm.at[pl.ds(base, _B_PER_W)])


@functools.cache
def _gather_kernel():
    return pl.kernel(
        _sc_gather,
        mesh=plsc.VectorSubcoreMesh(core_axis_name="c", subcore_axis_name="s"),
        out_type=jax.ShapeDtypeStruct((BATCH, EMBED_DIM), jnp.float32),
        scratch_types=[
            pltpu.VMEM((_B_PER_W,), jnp.int32),
            pltpu.VMEM((_B_PER_W, EMBED_DIM), jnp.float32),
            pltpu.SemaphoreType.DMA,
        ],
        compiler_params=pltpu.CompilerParams(use_tc_tiling_on_sc=False),
    )


_VBLK = 2048


def _proj_body(latent_ref, w_ref, b_ref, out_ref):
    out_ref[...] = lax.dot_general(
        latent_ref[...], w_ref[...],
        dimension_numbers=(((1,), (1,)), ((), ())),
        preferred_element_type=jnp.float32,
    ) + b_ref[...]


def _project(latent, W, b2d):
    grid = (pl.cdiv(VOCAB, _VBLK),)
    return pl.pallas_call(
        _proj_body,
        grid=grid,
        in_specs=[
            pl.BlockSpec((BATCH, EMBED_DIM), lambda j: (0, 0)),
            pl.BlockSpec((_VBLK, EMBED_DIM), lambda j: (j, 0)),
            pl.BlockSpec((1, _VBLK), lambda j: (0, j)),
        ],
        out_specs=pl.BlockSpec((BATCH, _VBLK), lambda j: (0, j)),
        out_shape=jax.ShapeDtypeStruct((BATCH, VOCAB), jnp.float32),
        compiler_params=pltpu.CompilerParams(
            dimension_semantics=("arbitrary",),
        ),
    )(latent, W, b2d)


def kernel(inputs, emb_table, W, b):
    idx = inputs.astype(jnp.int32)
    latent = jnp.take(emb_table, idx, axis=0)
    return _project(latent, W, b.reshape(1, VOCAB))


# speedup vs baseline: 2.9167x; 2.8393x over previous
import functools

import jax
import jax.numpy as jnp
from jax import lax
from jax.experimental import pallas as pl
from jax.experimental.pallas import tpu as pltpu
from jax.experimental.pallas import tpu_sc as plsc

VOCAB = 100000
EMBED_DIM = 16
BATCH = 1024


_NC = 2
_NS = 16
_NW = _NC * _NS
_B_PER_W = BATCH // _NW


def _sc_gather(idx_hbm, table_hbm, out_hbm, idx_v, rows_v, sem):
    wid = lax.axis_index("s") * _NC + lax.axis_index("c")
    base = wid * _B_PER_W
    pltpu.sync_copy(idx_hbm.at[pl.ds(base, _B_PER_W)], idx_v)
    pltpu.async_copy(table_hbm.at[idx_v], rows_v, sem).wait()
    pltpu.sync_copy(rows_v, out_hbm.at[pl.ds(base, _B_PER_W)])


@functools.cache
def _gather_kernel():
    return pl.kernel(
        _sc_gather,
        mesh=plsc.VectorSubcoreMesh(core_axis_name="c", subcore_axis_name="s"),
        out_type=jax.ShapeDtypeStruct((BATCH, EMBED_DIM), jnp.float32),
        scratch_types=[
            pltpu.VMEM((_B_PER_W,), jnp.int32),
            pltpu.VMEM((_B_PER_W, EMBED_DIM), jnp.float32),
            pltpu.SemaphoreType.DMA,
        ],
        compiler_params=pltpu.CompilerParams(use_tc_tiling_on_sc=False),
    )


_K = EMBED_DIM + 1
_VBLK = 2048


def _proj_body(wt_ref, lat_ref, out_ref):
    out_ref[...] = lax.dot_general(
        wt_ref[...], lat_ref[...],
        dimension_numbers=(((0,), (0,)), ((), ())),
        preferred_element_type=jnp.float32,
    )


def _project(wt_aug, lat_aug):
    grid = (pl.cdiv(VOCAB, _VBLK),)
    return pl.pallas_call(
        _proj_body,
        grid=grid,
        in_specs=[
            pl.BlockSpec((_K, _VBLK), lambda j: (0, j)),
            pl.BlockSpec((_K, BATCH), lambda j: (0, 0)),
        ],
        out_specs=pl.BlockSpec((_VBLK, BATCH), lambda j: (j, 0)),
        out_shape=jax.ShapeDtypeStruct((VOCAB, BATCH), jnp.float32),
        compiler_params=pltpu.CompilerParams(
            dimension_semantics=("arbitrary",),
        ),
    )(wt_aug, lat_aug)


def kernel(inputs, emb_table, W, b):
    idx = inputs.astype(jnp.int32)
    latent = _gather_kernel()(idx, emb_table)
    lat_aug = jnp.concatenate(
        [latent.T, jnp.ones((1, BATCH), jnp.float32)], axis=0)
    wt_aug = jnp.concatenate([W.T, b[None, :]], axis=0)
    return _project(wt_aug, lat_aug).T


# speedup vs baseline: 3.8248x; 1.3113x over previous
import jax
import jax.numpy as jnp
from jax import lax
from jax.experimental import pallas as pl
from jax.experimental.pallas import tpu as pltpu

VOCAB = 100000
EMBED_DIM = 16
BATCH = 1024

_K = EMBED_DIM + 1
_VBLK = 2048
_LANES = 128


def _proj_body(idx_ref, tt_ref, wt_ref, out_ref, lat_ref):
    @pl.when(pl.program_id(0) == 0)
    def _gather():
        lat_ref[EMBED_DIM:_K, :] = jnp.ones((_K - EMBED_DIM, BATCH), jnp.float32)
        lane_ids = lax.broadcasted_iota(jnp.int32, (EMBED_DIM, _LANES), 1)

        def tile_body(t, _):
            def lane_body(i, acc):
                c = idx_ref[t * _LANES + i]
                cb = pl.multiple_of((c // _LANES) * _LANES, _LANES)
                tile = tt_ref[:, pl.ds(cb, _LANES)]
                rolled = pltpu.roll(tile, i - (c - cb), axis=1)
                return jnp.where(lane_ids == i, rolled, acc)

            acc = lax.fori_loop(
                0, _LANES, lane_body,
                jnp.zeros((EMBED_DIM, _LANES), jnp.float32), unroll=8)
            lat_ref[0:EMBED_DIM, pl.ds(pl.multiple_of(t * _LANES, _LANES), _LANES)] = acc
            return 0

        lax.fori_loop(0, BATCH // _LANES, tile_body, 0)

    out_ref[...] = lax.dot_general(
        wt_ref[...], lat_ref[...],
        dimension_numbers=(((0,), (0,)), ((), ())),
        preferred_element_type=jnp.float32,
    )


def _project(idx, tableT, wt_aug):
    grid = (pl.cdiv(VOCAB, _VBLK),)
    return pl.pallas_call(
        _proj_body,
        grid=grid,
        in_specs=[
            pl.BlockSpec(memory_space=pltpu.SMEM),
            pl.BlockSpec((EMBED_DIM, VOCAB), lambda j: (0, 0)),
            pl.BlockSpec((_K, _VBLK), lambda j: (0, j)),
        ],
        out_specs=pl.BlockSpec((_VBLK, BATCH), lambda j: (j, 0)),
        out_shape=jax.ShapeDtypeStruct((VOCAB, BATCH), jnp.float32),
        scratch_shapes=[pltpu.VMEM((_K, BATCH), jnp.float32)],
        compiler_params=pltpu.CompilerParams(
            dimension_semantics=("arbitrary",),
            vmem_limit_bytes=100 * 1024 * 1024,
        ),
    )(idx, tableT, wt_aug)


def kernel(inputs, emb_table, W, b):
    idx = inputs.astype(jnp.int32)
    wt_aug = jnp.concatenate([W.T, b[None, :]], axis=0)
    return _project(idx, emb_table.T, wt_aug).T


# speedup vs baseline: 4.0097x; 1.0483x over previous
import jax
import jax.numpy as jnp
from jax import lax
from jax.experimental import pallas as pl
from jax.experimental.pallas import tpu as pltpu

VOCAB = 100000
EMBED_DIM = 16
BATCH = 1024

_VBLK = 2048
_LANES = 128


def _proj_body(idx_ref, tt_ref, wt_ref, b_ref, out_ref, lat_ref):
    @pl.when(pl.program_id(0) == 0)
    def _gather():
        lane_ids = lax.broadcasted_iota(jnp.int32, (EMBED_DIM, _LANES), 1)

        def tile_body(t, _):
            def lane_body(i, acc):
                c = idx_ref[t * _LANES + i]
                cb = pl.multiple_of((c // _LANES) * _LANES, _LANES)
                tile = tt_ref[:, pl.ds(cb, _LANES)]
                rolled = pltpu.roll(tile, i - (c - cb), axis=1)
                return jnp.where(lane_ids == i, rolled, acc)

            acc = lax.fori_loop(
                0, _LANES, lane_body,
                jnp.zeros((EMBED_DIM, _LANES), jnp.float32), unroll=8)
            lat_ref[:, pl.ds(pl.multiple_of(t * _LANES, _LANES), _LANES)] = acc
            return 0

        lax.fori_loop(0, BATCH // _LANES, tile_body, 0)

    bcol = lax.dot_general(
        b_ref[...], jnp.ones((1, 1), jnp.float32),
        dimension_numbers=(((0,), (0,)), ((), ())),
        preferred_element_type=jnp.float32,
    )
    out_ref[...] = lax.dot_general(
        wt_ref[...], lat_ref[...],
        dimension_numbers=(((0,), (0,)), ((), ())),
        preferred_element_type=jnp.float32,
    ) + bcol


def _project(idx, tableT, wt, brow):
    grid = (pl.cdiv(VOCAB, _VBLK),)
    return pl.pallas_call(
        _proj_body,
        grid=grid,
        in_specs=[
            pl.BlockSpec(memory_space=pltpu.SMEM),
            pl.BlockSpec((EMBED_DIM, VOCAB), lambda j: (0, 0)),
            pl.BlockSpec((EMBED_DIM, _VBLK), lambda j: (0, j)),
            pl.BlockSpec((1, _VBLK), lambda j: (0, j)),
        ],
        out_specs=pl.BlockSpec((_VBLK, BATCH), lambda j: (j, 0)),
        out_shape=jax.ShapeDtypeStruct((VOCAB, BATCH), jnp.float32),
        scratch_shapes=[pltpu.VMEM((EMBED_DIM, BATCH), jnp.float32)],
        compiler_params=pltpu.CompilerParams(
            dimension_semantics=("arbitrary",),
            vmem_limit_bytes=100 * 1024 * 1024,
        ),
    )(idx, tableT, wt, brow)


def kernel(inputs, emb_table, W, b):
    idx = inputs.astype(jnp.int32)
    return _project(idx, emb_table.T, W.T, b[None, :]).T
